# Initial kernel scaffold; baseline (speedup 1.0000x reference)
import jax
import jax.numpy as jnp
from jax.experimental import pallas as pl


def kernel(proc, meas, drug, proc_table, meas_table, drug_table, fuse_W, fuse_b, head_W, head_b):
    raise NotImplementedError("write your pallas kernel here")


# speedup vs baseline: 3.3346x; 3.3346x over previous
import functools

import jax
import jax.numpy as jnp
from jax import lax
from jax.experimental import pallas as pl
from jax.experimental.pallas import tpu as pltpu
from jax.experimental.pallas import tpu_sc as plsc

B = 16384
H = 128

_INFO = plsc.get_sparse_core_info()
_NC, _NS = _INFO.num_cores, _INFO.num_subcores
_NW = _NC * _NS
_BPW = B // _NW
_CHUNK = 128
_NCHUNK = _BPW // _CHUNK


def _sc_gather(proc_table, meas_table, drug_table, proc, meas, drug):
    mesh = plsc.VectorSubcoreMesh(core_axis_name="c", subcore_axis_name="s")

    @functools.partial(
        pl.kernel,
        mesh=mesh,
        out_type=[jax.ShapeDtypeStruct((B, H), jnp.float32)] * 3,
        scratch_types=[
            pltpu.VMEM((_NCHUNK, _CHUNK), jnp.int32),
            pltpu.VMEM((2, _CHUNK, H), jnp.float32),
            pltpu.SemaphoreType.DMA,
            pltpu.SemaphoreType.DMA,
        ],
    )
    def k(pt, mt, dt, pi, mi, di, o0, o1, o2, idx_v, rows_v, sem0, sem1):
        wid = lax.axis_index("s") * _NC + lax.axis_index("c")
        base = wid * _BPW
        sems = (sem0, sem1)
        for t_ref, i_ref, o_ref in ((pt, pi, o0), (mt, mi, o1), (dt, di, o2)):
            for j in range(_NCHUNK):
                pltpu.sync_copy(
                    i_ref.at[pl.ds(base + j * _CHUNK, _CHUNK)], idx_v.at[j]
                )
            cps = []
            for j in range(_NCHUNK):
                b = j % 2
                cp = pltpu.async_copy(t_ref.at[idx_v.at[j]], rows_v.at[b], sems[b])
                cps.append(cp)
                if j >= 1:
                    cps[j - 1].wait()
                    pltpu.sync_copy(
                        rows_v.at[(j - 1) % 2],
                        o_ref.at[pl.ds(base + (j - 1) * _CHUNK, _CHUNK)],
                    )
            cps[-1].wait()
            pltpu.sync_copy(
                rows_v.at[(_NCHUNK - 1) % 2],
                o_ref.at[pl.ds(base + (_NCHUNK - 1) * _CHUNK, _CHUNK)],
            )

    return k(proc_table, meas_table, drug_table, proc, meas, drug)


_BM = 2048


def _mlp_body(hp_ref, hm_ref, hd_ref, fw_ref, fb_ref, hw_ref, hb_ref, o_ref):
    acc = jnp.dot(hp_ref[...], fw_ref[0:H, :], preferred_element_type=jnp.float32)
    acc += jnp.dot(hm_ref[...], fw_ref[H:2 * H, :], preferred_element_type=jnp.float32)
    acc += jnp.dot(hd_ref[...], fw_ref[2 * H:3 * H, :], preferred_element_type=jnp.float32)
    h = jnp.maximum(acc + fb_ref[...], 0.0)
    o_ref[...] = jnp.dot(h, hw_ref[...], preferred_element_type=jnp.float32) + hb_ref[...]


def _tc_mlp(hp, hm, hd, fuse_W, fuse_b, head_W, head_b):
    grid = (B // _BM,)
    return pl.pallas_call(
        _mlp_body,
        grid=grid,
        in_specs=[
            pl.BlockSpec((_BM, H), lambda i: (i, 0)),
            pl.BlockSpec((_BM, H), lambda i: (i, 0)),
            pl.BlockSpec((_BM, H), lambda i: (i, 0)),
            pl.BlockSpec((3 * H, H), lambda i: (0, 0)),
            pl.BlockSpec((H,), lambda i: (0,)),
            pl.BlockSpec((H, 4), lambda i: (0, 0)),
            pl.BlockSpec((4,), lambda i: (0,)),
        ],
        out_specs=pl.BlockSpec((_BM, 4), lambda i: (i, 0)),
        out_shape=jax.ShapeDtypeStruct((B, 4), jnp.float32),
    )(hp, hm, hd, fuse_W, fuse_b, head_W, head_b)


def kernel(proc, meas, drug, proc_table, meas_table, drug_table,
           fuse_W, fuse_b, head_W, head_b):
    hp, hm, hd = _sc_gather(proc_table, meas_table, drug_table,
                            proc.astype(jnp.int32), meas.astype(jnp.int32),
                            drug.astype(jnp.int32))
    return _tc_mlp(hp, hm, hd, fuse_W, fuse_b, head_W, head_b)


# speedup vs baseline: 4.0992x; 1.2293x over previous
import functools

import jax
import jax.numpy as jnp
from jax import lax
from jax.experimental import pallas as pl
from jax.experimental.pallas import tpu as pltpu
from jax.experimental.pallas import tpu_sc as plsc

B = 16384
H = 128
_NSLICE = 2
_SB = B // _NSLICE

_INFO = plsc.get_sparse_core_info()
_NC, _NS = _INFO.num_cores, _INFO.num_subcores
_NW = _NC * _NS
_BPW = _SB // _NW
_CHUNK = 128
_NCHUNK = _BPW // _CHUNK
_NT = 3 * _NCHUNK
_NBUF = 4 if _NT >= 4 else _NT
_AHEAD = 3 if _NT >= 3 else _NT


def _sc_gather(proc_table, meas_table, drug_table, proc, meas, drug):
    mesh = plsc.VectorSubcoreMesh(core_axis_name="c", subcore_axis_name="s")

    @functools.partial(
        pl.kernel,
        mesh=mesh,
        out_type=[jax.ShapeDtypeStruct((_SB, H), jnp.float32)] * 3,
        scratch_types=[
            pltpu.VMEM((3, _NCHUNK, _CHUNK), jnp.int32),
            pltpu.VMEM((_NBUF, _CHUNK, H), jnp.float32),
        ]
        + [pltpu.SemaphoreType.DMA] * (3 + 2 * _NBUF),
    )
    def k(pt, mt, dt, pi, mi, di, o0, o1, o2, idx_v, rows_v, *sems):
        isem = sems[0:3]
        gsem = sems[3:3 + _NBUF]
        wsem = sems[3 + _NBUF:3 + 2 * _NBUF]
        wid = lax.axis_index("s") * _NC + lax.axis_index("c")
        base = wid * _BPW
        tabs = (pt, mt, dt)
        irefs = (pi, mi, di)
        orefs = (o0, o1, o2)

        icp = [
            pltpu.async_copy(
                irefs[t].at[pl.ds(wid * _NCHUNK, _NCHUNK)], idx_v.at[t], isem[t]
            )
            for t in range(3)
        ]
        idx_ready = [False, False, False]
        gcp = [None] * _NT
        wcp = [None] * _NT

        def fire(c):
            t, j = divmod(c, _NCHUNK)
            if not idx_ready[t]:
                icp[t].wait()
                idx_ready[t] = True
            if c >= _NBUF:
                wcp[c - _NBUF].wait()
            b = c % _NBUF
            gcp[c] = pltpu.async_copy(
                tabs[t].at[idx_v.at[t].at[j]], rows_v.at[b], gsem[b]
            )

        for c in range(min(_AHEAD, _NT)):
            fire(c)
        for c in range(_NT):
            t, j = divmod(c, _NCHUNK)
            gcp[c].wait()
            wcp[c] = pltpu.async_copy(
                rows_v.at[c % _NBUF],
                orefs[t].at[pl.ds(base + j * _CHUNK, _CHUNK)],
                wsem[c % _NBUF],
            )
            if c + _AHEAD < _NT:
                fire(c + _AHEAD)
        for c in range(max(0, _NT - _NBUF), _NT):
            wcp[c].wait()

    return k(proc_table, meas_table, drug_table, proc, meas, drug)


_BM = 2048


def _mlp_body(hp_ref, hm_ref, hd_ref, fw_ref, fb_ref, hw_ref, hb_ref, o_ref):
    acc = jnp.dot(hp_ref[...], fw_ref[0:H, :], preferred_element_type=jnp.float32)
    acc += jnp.dot(hm_ref[...], fw_ref[H:2 * H, :], preferred_element_type=jnp.float32)
    acc += jnp.dot(hd_ref[...], fw_ref[2 * H:3 * H, :], preferred_element_type=jnp.float32)
    h = jnp.maximum(acc + fb_ref[...], 0.0)
    ot = lax.dot_general(hw_ref[...], h, (((0,), (1,)), ((), ())),
                         preferred_element_type=jnp.float32)
    o_ref[...] = ot + hb_ref[...][:, None]


def _tc_mlp(hp, hm, hd, fuse_W, fuse_b, head_W, head_b):
    grid = (_SB // _BM,)
    return pl.pallas_call(
        _mlp_body,
        grid=grid,
        in_specs=[
            pl.BlockSpec((_BM, H), lambda i: (i, 0)),
            pl.BlockSpec((_BM, H), lambda i: (i, 0)),
            pl.BlockSpec((_BM, H), lambda i: (i, 0)),
            pl.BlockSpec((3 * H, H), lambda i: (0, 0)),
            pl.BlockSpec((H,), lambda i: (0,)),
            pl.BlockSpec((H, 4), lambda i: (0, 0)),
            pl.BlockSpec((4,), lambda i: (0,)),
        ],
        out_specs=pl.BlockSpec((4, _BM), lambda i: (0, i)),
        out_shape=jax.ShapeDtypeStruct((4, _SB), jnp.float32),
    )(hp, hm, hd, fuse_W, fuse_b, head_W, head_b)


def kernel(proc, meas, drug, proc_table, meas_table, drug_table,
           fuse_W, fuse_b, head_W, head_b):
    proc = proc.astype(jnp.int32).reshape(B // _CHUNK, _CHUNK)
    meas = meas.astype(jnp.int32).reshape(B // _CHUNK, _CHUNK)
    drug = drug.astype(jnp.int32).reshape(B // _CHUNK, _CHUNK)
    nrows = _SB // _CHUNK
    outs = []
    for s in range(_NSLICE):
        hp, hm, hd = _sc_gather(
            proc_table, meas_table, drug_table,
            lax.dynamic_slice(proc, (s * nrows, 0), (nrows, _CHUNK)),
            lax.dynamic_slice(meas, (s * nrows, 0), (nrows, _CHUNK)),
            lax.dynamic_slice(drug, (s * nrows, 0), (nrows, _CHUNK)))
        outs.append(_tc_mlp(hp, hm, hd, fuse_W, fuse_b, head_W, head_b))
    out_t = outs[0] if _NSLICE == 1 else jnp.concatenate(outs, axis=1)
    return out_t.T


# speedup vs baseline: 4.2768x; 1.0433x over previous
import functools

import jax
import jax.numpy as jnp
from jax import lax
from jax.experimental import pallas as pl
from jax.experimental.pallas import tpu as pltpu
from jax.experimental.pallas import tpu_sc as plsc

B = 16384
H = 128
_NSLICE = 2
_SB = B // _NSLICE

_INFO = plsc.get_sparse_core_info()
_NC, _NS = _INFO.num_cores, _INFO.num_subcores
_NW = _NC * _NS
_BPW = _SB // _NW
_CHUNK = 128
_NCHUNK = _BPW // _CHUNK
_NT = 3 * _NCHUNK
_NBUF = min(_NT, 6)
_AHEAD = _NBUF


def _sc_gather(proc_table, meas_table, drug_table, proc, meas, drug, slice_id):
    mesh = plsc.VectorSubcoreMesh(core_axis_name="c", subcore_axis_name="s")

    @functools.partial(
        pl.kernel,
        mesh=mesh,
        out_type=[jax.ShapeDtypeStruct((_SB, H), jnp.float32)] * 3,
        scratch_types=[
            pltpu.VMEM((3, _NCHUNK, _CHUNK), jnp.int32),
            pltpu.VMEM((_NBUF, _CHUNK, H), jnp.float32),
        ]
        + [pltpu.SemaphoreType.DMA] * (3 + 2 * _NBUF),
    )
    def k(pt, mt, dt, pi, mi, di, o0, o1, o2, idx_v, rows_v, *sems):
        isem = sems[0:3]
        gsem = sems[3:3 + _NBUF]
        wsem = sems[3 + _NBUF:3 + 2 * _NBUF]
        wid = lax.axis_index("s") * _NC + lax.axis_index("c")
        base = wid * _BPW
        irow = slice_id * (_SB // _CHUNK) + wid * _NCHUNK
        tabs = (pt, mt, dt)
        irefs = (pi, mi, di)
        orefs = (o0, o1, o2)

        icp = [
            pltpu.async_copy(
                irefs[t].at[pl.ds(irow, _NCHUNK)], idx_v.at[t], isem[t]
            )
            for t in range(3)
        ]
        idx_ready = [False, False, False]
        gcp = [None] * _NT
        wcp = [None] * _NT

        def fire(c):
            t, j = divmod(c, _NCHUNK)
            if not idx_ready[t]:
                icp[t].wait()
                idx_ready[t] = True
            if c >= _NBUF:
                wcp[c - _NBUF].wait()
            b = c % _NBUF
            gcp[c] = pltpu.async_copy(
                tabs[t].at[idx_v.at[t].at[j]], rows_v.at[b], gsem[b]
            )

        for c in range(min(_AHEAD, _NT)):
            fire(c)
        for c in range(_NT):
            t, j = divmod(c, _NCHUNK)
            gcp[c].wait()
            wcp[c] = pltpu.async_copy(
                rows_v.at[c % _NBUF],
                orefs[t].at[pl.ds(base + j * _CHUNK, _CHUNK)],
                wsem[c % _NBUF],
            )
            if c + _AHEAD < _NT:
                fire(c + _AHEAD)
        for c in range(max(0, _NT - _NBUF), _NT):
            wcp[c].wait()

    return k(proc_table, meas_table, drug_table, proc, meas, drug)


_BM = 2048


def _mlp_body(hp_ref, hm_ref, hd_ref, fw_ref, fb_ref, hw_ref, hb_ref, o_ref):
    acc = jnp.dot(hp_ref[...], fw_ref[0:H, :], preferred_element_type=jnp.float32)
    acc += jnp.dot(hm_ref[...], fw_ref[H:2 * H, :], preferred_element_type=jnp.float32)
    acc += jnp.dot(hd_ref[...], fw_ref[2 * H:3 * H, :], preferred_element_type=jnp.float32)
    h = jnp.maximum(acc + fb_ref[...], 0.0)
    ot = lax.dot_general(hw_ref[...], h, (((0,), (1,)), ((), ())),
                         preferred_element_type=jnp.float32)
    o_ref[...] = ot + hb_ref[...][:, None]


def _tc_mlp(hp, hm, hd, fuse_W, fuse_b, head_W, head_b):
    grid = (_SB // _BM,)
    return pl.pallas_call(
        _mlp_body,
        grid=grid,
        in_specs=[
            pl.BlockSpec((_BM, H), lambda i: (i, 0)),
            pl.BlockSpec((_BM, H), lambda i: (i, 0)),
            pl.BlockSpec((_BM, H), lambda i: (i, 0)),
            pl.BlockSpec((3 * H, H), lambda i: (0, 0)),
            pl.BlockSpec((H,), lambda i: (0,)),
            pl.BlockSpec((H, 4), lambda i: (0, 0)),
            pl.BlockSpec((4,), lambda i: (0,)),
        ],
        out_specs=pl.BlockSpec((4, _BM), lambda i: (0, i)),
        out_shape=jax.ShapeDtypeStruct((4, _SB), jnp.float32),
    )(hp, hm, hd, fuse_W, fuse_b, head_W, head_b)


def kernel(proc, meas, drug, proc_table, meas_table, drug_table,
           fuse_W, fuse_b, head_W, head_b):
    proc = proc.astype(jnp.int32).reshape(B // _CHUNK, _CHUNK)
    meas = meas.astype(jnp.int32).reshape(B // _CHUNK, _CHUNK)
    drug = drug.astype(jnp.int32).reshape(B // _CHUNK, _CHUNK)
    outs = []
    for s in range(_NSLICE):
        hp, hm, hd = _sc_gather(proc_table, meas_table, drug_table,
                                proc, meas, drug, s)
        outs.append(_tc_mlp(hp, hm, hd, fuse_W, fuse_b, head_W, head_b))
    out_t = outs[0] if _NSLICE == 1 else jnp.concatenate(outs, axis=1)
    return out_t.T
